# Initial kernel scaffold; baseline (speedup 1.0000x reference)
import jax
import jax.numpy as jnp
from jax.experimental import pallas as pl


def kernel(pos):
    raise NotImplementedError("write your pallas kernel here")


# speedup vs baseline: 5.3346x; 5.3346x over previous
import functools

import jax
import jax.numpy as jnp
from jax.experimental import pallas as pl
from jax.experimental.pallas import tpu as pltpu

_N = 10000
_N_KEEP = 7000
_K = 50
_NUM_BINS = 5
_CUTOFF = 5.0
_RADIUS = 0.75

_NPAD = 7040
_R = 128
_NBLK = _NPAD // _R
_PADV = 1.0e20


def _knn_body(keys_ref, q_ref, sq_ref, idx_ref, val_ref, d2s, e2s):
    i = pl.program_id(0)
    kx = keys_ref[0:1, :]
    ky = keys_ref[1:2, :]
    kz = keys_ref[2:3, :]
    qx = q_ref[:, 0:1]
    qy = q_ref[:, 1:2]
    qz = q_ref[:, 2:3]
    sqq = q_ref[:, 3:4]
    sqk = sq_ref[0:1, :]
    mm = jnp.dot(q_ref[...], keys_ref[...],
                 preferred_element_type=jnp.float32)
    d2 = (sqq + sqk) - 2.0 * mm
    dx = qx - kx
    dy = qy - ky
    dz = qz - kz
    e2 = dx * dx + dy * dy + dz * dz
    row = i * _R + jax.lax.broadcasted_iota(jnp.int32, (_R, 1), 0)
    col = jax.lax.broadcasted_iota(jnp.int32, (_R, _NPAD), 1)
    d2 = jnp.where(col == row, jnp.inf, d2)
    d2s[...] = d2
    e2s[...] = e2
    idx_ref[...] = jnp.zeros((_R, 128), jnp.int32)
    val_ref[...] = jnp.zeros((_R, 128), jnp.float32)
    lane = jax.lax.broadcasted_iota(jnp.int32, (_R, 128), 1)

    def body(k, _):
        d2c = d2s[...]
        m = jnp.min(d2c, axis=1, keepdims=True)
        sel = jnp.min(jnp.where(d2c == m, col, _NPAD), axis=1, keepdims=True)
        ev = jnp.min(jnp.where(col == sel, e2s[...], jnp.inf),
                     axis=1, keepdims=True)
        val_ref[...] = jnp.where(lane == k, ev, val_ref[...])
        idx_ref[...] = jnp.where(lane == k, sel, idx_ref[...])
        d2s[...] = jnp.where(col == sel, jnp.inf, d2c)
        return 0

    jax.lax.fori_loop(0, _K, body, 0)


def _attr_body(d2_ref, attr_ref):
    centers = [_CUTOFF / (_NUM_BINS - 1) * c for c in range(_NUM_BINS)]
    sigma = centers[1] - centers[0]
    inv = 1.0 / (2.0 * sigma * sigma)
    dist = jnp.sqrt(d2_ref[...] + 1e-12)
    for c in range(_NUM_BINS):
        diff = dist - centers[c]
        attr_ref[c] = jnp.exp(-(diff * diff) * inv)
    for c in range(_NUM_BINS, 8):
        attr_ref[c] = jnp.zeros_like(dist)


def kernel(pos):
    base = jax.random.key(1)
    k1 = jax.random.fold_in(base, 0)
    k2 = jax.random.fold_in(base, 1)
    k3 = jax.random.fold_in(base, 2)
    scores = jax.random.uniform(k1, (_N,))
    keep_idx = jnp.argsort(scores)[:_N_KEEP]
    p = jnp.take(pos, keep_idx, axis=0)
    dirs = jax.random.normal(k2, (_N_KEEP, 3), dtype=jnp.float32)
    dirs = dirs / (jnp.linalg.norm(dirs, axis=1, keepdims=True) + 1e-12)
    u = jax.random.uniform(k3, (_N_KEEP, 1), dtype=jnp.float32)
    p = p + dirs * _RADIUS * (u ** (1.0 / 3.0))

    sq = jnp.sum(p * p, axis=1)

    p_pad = jnp.pad(p, ((0, _NPAD - _N_KEEP), (0, 0)))
    q_arr = jnp.concatenate(
        [p_pad, jnp.pad(sq, (0, _NPAD - _N_KEEP))[:, None],
         jnp.zeros((_NPAD, 128 - 4), jnp.float32)], axis=1)
    keys = jnp.pad(p_pad.T, ((0, 128 - 3), (0, 0)))
    sq_in = jnp.pad(
        jnp.pad(sq, (0, _NPAD - _N_KEEP),
                constant_values=jnp.inf)[None, :], ((0, 7), (0, 0)))

    idx_out, val_out = pl.pallas_call(
        _knn_body,
        grid=(_NBLK,),
        in_specs=[
            pl.BlockSpec((128, _NPAD), lambda i: (0, 0)),
            pl.BlockSpec((_R, 128), lambda i: (i, 0)),
            pl.BlockSpec((8, _NPAD), lambda i: (0, 0)),
        ],
        out_specs=[
            pl.BlockSpec((_R, 128), lambda i: (i, 0)),
            pl.BlockSpec((_R, 128), lambda i: (i, 0)),
        ],
        out_shape=[
            jax.ShapeDtypeStruct((_NPAD, 128), jnp.int32),
            jax.ShapeDtypeStruct((_NPAD, 128), jnp.float32),
        ],
        scratch_shapes=[pltpu.VMEM((_R, _NPAD), jnp.float32),
                        pltpu.VMEM((_R, _NPAD), jnp.float32)],
        compiler_params=pltpu.CompilerParams(
            dimension_semantics=("arbitrary",)),
    )(keys, q_arr, sq_in)

    attr_out = pl.pallas_call(
        _attr_body,
        grid=(_NBLK,),
        in_specs=[pl.BlockSpec((_R, 128), lambda i: (i, 0))],
        out_specs=pl.BlockSpec((8, _R, 128), lambda i: (0, i, 0)),
        out_shape=jax.ShapeDtypeStruct((8, _NPAD, 128), jnp.float32),
    )(val_out)

    nbr = idx_out[:_N_KEEP, :_K]
    src = nbr.reshape(-1)
    dst = jnp.repeat(jnp.arange(_N_KEEP), _K)
    row = jnp.concatenate([src, dst])
    col = jnp.concatenate([dst, src])
    edge_index = jnp.stack([row, col], axis=0)

    attr_half = jnp.transpose(
        attr_out[:_NUM_BINS, :_N_KEEP, :_K], (1, 2, 0)).reshape(-1, _NUM_BINS)
    edge_attr = jnp.concatenate([attr_half, attr_half], axis=0)
    return edge_index, edge_attr


# speedup vs baseline: 6.7892x; 1.2727x over previous
import functools

import jax
import jax.numpy as jnp
from jax import lax
from jax.experimental import pallas as pl
from jax.experimental.pallas import tpu as pltpu
from jax.experimental.pallas import tpu_sc as plsc

_N = 10000
_N_KEEP = 7000
_K = 50
_NUM_BINS = 5
_CUTOFF = 5.0
_RADIUS = 0.75

_NPAD = 7040
_R = 128
_NBLK = _NPAD // _R
_CAP = 128
_NSUB = 32
_ROWS_W = _NPAD // _NSUB
_NCHUNK = 110


def _d2_body(keys_ref, q_ref, sq_ref, d2_ref, tau_ref, cm_s):
    i = pl.program_id(0)
    sqq = q_ref[:, 3:4]
    sqk = sq_ref[0:1, :]
    mm = jnp.dot(q_ref[...], keys_ref[...],
                 preferred_element_type=jnp.float32)
    d2 = (sqq + sqk) - 2.0 * mm
    row = i * _R + lax.broadcasted_iota(jnp.int32, (_R, 1), 0)
    col = lax.broadcasted_iota(jnp.int32, (_R, _NPAD), 1)
    d2 = jnp.where(col == row, jnp.inf, d2)
    d2_ref[...] = d2
    lane = lax.broadcasted_iota(jnp.int32, (_R, 128), 1)
    cm = jnp.full((_R, 128), jnp.inf, jnp.float32)
    for c in range(_NCHUNK):
        mn = jnp.min(d2_ref[:, 64 * c:64 * (c + 1)], axis=1, keepdims=True)
        cm = jnp.where(lane == c, mn, cm)
    cm_s[...] = cm

    def pop(k, _):
        cmc = cm_s[...]
        m = jnp.min(cmc, axis=1, keepdims=True)
        sel = jnp.min(jnp.where(cmc == m, lane, 128), axis=1, keepdims=True)
        cm_s[...] = jnp.where(lane == sel, jnp.inf, cmc)
        return 0

    lax.fori_loop(0, _K - 1, pop, 0)
    tau = jnp.min(cm_s[...], axis=1, keepdims=True)
    tau_ref[...] = jnp.broadcast_to(tau, (_R, 128))


def _sc_compact(d2_hbm, tau_hbm, kx_hbm, ky_hbm, kz_hbm,
                od2_hbm, ox_hbm, oy_hbm, oz_hbm, oi_hbm,
                kxb, kyb, kzb, taub, rowb8, candd, candi,
                od2b, oxb, oyb, ozb, oib):
    wid = lax.axis_index("s") * 2 + lax.axis_index("c")
    g0 = 27 * wid + jnp.minimum(wid, 16)
    ng = jnp.where(wid < 16, 28, 27)
    pltpu.sync_copy(kx_hbm, kxb)
    pltpu.sync_copy(ky_hbm, kyb)
    pltpu.sync_copy(kz_hbm, kzb)
    pltpu.sync_copy(tau_hbm, taub)
    iota16 = lax.broadcasted_iota(jnp.int32, (16,), 0)

    def group_body(gl, _):
        g = g0 + gl
        pltpu.sync_copy(d2_hbm.at[pl.ds(g * 8, 8)], rowb8)
        for rr in range(8):
            r = g * 8 + rr
            tauv = plsc.load_gather(taub, [jnp.full((16,), r, jnp.int32)])
            for s in range(_CAP // 16):
                candd[pl.ds(16 * s, 16)] = jnp.full((16,), jnp.inf,
                                                    jnp.float32)
                candi[pl.ds(16 * s, 16)] = jnp.zeros((16,), jnp.int32)

            def chunk(c, off):
                d2v = rowb8[rr, pl.ds(c * 16, 16)]
                mask = d2v <= tauv
                cnt = jnp.sum(mask.astype(jnp.int32))
                iv = iota16 + c * 16
                plsc.store_compressed(candd.at[pl.ds(off, 16)], d2v,
                                      mask=mask)
                plsc.store_compressed(candi.at[pl.ds(off, 16)], iv,
                                      mask=mask)
                return jnp.minimum(off + cnt, _CAP)

            lax.fori_loop(0, _NPAD // 16, chunk, 0)
            for s in range(_CAP // 16):
                sl = pl.ds(16 * s, 16)
                ivv = candi[sl]
                od2b[rr, sl] = candd[sl]
                oib[rr, sl] = ivv
                oxb[rr, sl] = plsc.load_gather(kxb, [ivv])
                oyb[rr, sl] = plsc.load_gather(kyb, [ivv])
                ozb[rr, sl] = plsc.load_gather(kzb, [ivv])
        sl8 = pl.ds(g * 8, 8)
        pltpu.sync_copy(od2b, od2_hbm.at[sl8])
        pltpu.sync_copy(oxb, ox_hbm.at[sl8])
        pltpu.sync_copy(oyb, oy_hbm.at[sl8])
        pltpu.sync_copy(ozb, oz_hbm.at[sl8])
        pltpu.sync_copy(oib, oi_hbm.at[sl8])
        return 0

    lax.fori_loop(0, ng, group_body, 0)


def _sel_body(cd_ref, cx_ref, cy_ref, cz_ref, ci_ref, q_ref,
              idx_ref, val_ref, d2s):
    qx = q_ref[:, 0:1]
    qy = q_ref[:, 1:2]
    qz = q_ref[:, 2:3]
    dx = qx - cx_ref[...]
    dy = qy - cy_ref[...]
    dz = qz - cz_ref[...]
    e2 = dx * dx + dy * dy + dz * dz
    d2s[...] = cd_ref[...]
    idx_ref[...] = jnp.zeros((_R, 128), jnp.int32)
    val_ref[...] = jnp.zeros((_R, 128), jnp.float32)
    lane = lax.broadcasted_iota(jnp.int32, (_R, _CAP), 1)

    def body(k, _):
        d2c = d2s[...]
        m = jnp.min(d2c, axis=1, keepdims=True)
        sel = jnp.min(jnp.where(d2c == m, lane, _CAP), axis=1, keepdims=True)
        hit = lane == sel
        ev = jnp.min(jnp.where(hit, e2, jnp.inf), axis=1, keepdims=True)
        gi = jnp.min(jnp.where(hit, ci_ref[...], _NPAD), axis=1, keepdims=True)
        val_ref[...] = jnp.where(lane == k, ev, val_ref[...])
        idx_ref[...] = jnp.where(lane == k, gi, idx_ref[...])
        d2s[...] = jnp.where(hit, jnp.inf, d2c)
        return 0

    lax.fori_loop(0, _K, body, 0)


def _attr_body(d2_ref, attr_ref):
    centers = [_CUTOFF / (_NUM_BINS - 1) * c for c in range(_NUM_BINS)]
    sigma = centers[1] - centers[0]
    inv = 1.0 / (2.0 * sigma * sigma)
    dist = jnp.sqrt(d2_ref[...] + 1e-12)
    for c in range(_NUM_BINS):
        diff = dist - centers[c]
        attr_ref[c] = jnp.exp(-(diff * diff) * inv)
    for c in range(_NUM_BINS, 8):
        attr_ref[c] = jnp.zeros_like(dist)


def kernel(pos):
    base = jax.random.key(1)
    k1 = jax.random.fold_in(base, 0)
    k2 = jax.random.fold_in(base, 1)
    k3 = jax.random.fold_in(base, 2)
    scores = jax.random.uniform(k1, (_N,))
    keep_idx = jnp.argsort(scores)[:_N_KEEP]
    p = jnp.take(pos, keep_idx, axis=0)
    dirs = jax.random.normal(k2, (_N_KEEP, 3), dtype=jnp.float32)
    dirs = dirs / (jnp.linalg.norm(dirs, axis=1, keepdims=True) + 1e-12)
    u = jax.random.uniform(k3, (_N_KEEP, 1), dtype=jnp.float32)
    p = p + dirs * _RADIUS * (u ** (1.0 / 3.0))

    sq = jnp.sum(p * p, axis=1)

    p_pad = jnp.pad(p, ((0, _NPAD - _N_KEEP), (0, 0)))
    q_arr = jnp.concatenate(
        [p_pad, jnp.pad(sq, (0, _NPAD - _N_KEEP))[:, None],
         jnp.zeros((_NPAD, 128 - 4), jnp.float32)], axis=1)
    keys = jnp.pad(p_pad.T, ((0, 128 - 3), (0, 0)))
    sq_in = jnp.pad(
        jnp.pad(sq, (0, _NPAD - _N_KEEP),
                constant_values=jnp.inf)[None, :], ((0, 7), (0, 0)))

    d2m, tau = pl.pallas_call(
        _d2_body,
        grid=(_NBLK,),
        in_specs=[
            pl.BlockSpec((128, _NPAD), lambda i: (0, 0)),
            pl.BlockSpec((_R, 128), lambda i: (i, 0)),
            pl.BlockSpec((8, _NPAD), lambda i: (0, 0)),
        ],
        out_specs=[
            pl.BlockSpec((_R, _NPAD), lambda i: (i, 0)),
            pl.BlockSpec((_R, 128), lambda i: (i, 0)),
        ],
        out_shape=[
            jax.ShapeDtypeStruct((_NPAD, _NPAD), jnp.float32),
            jax.ShapeDtypeStruct((_NPAD, 128), jnp.float32),
        ],
        scratch_shapes=[pltpu.VMEM((_R, 128), jnp.float32)],
        compiler_params=pltpu.CompilerParams(
            dimension_semantics=("arbitrary",)),
    )(keys, q_arr, sq_in)

    kx = p_pad[:, 0]
    ky = p_pad[:, 1]
    kz = p_pad[:, 2]
    tau1 = tau[:, 0]

    sc_fn = functools.partial(
        pl.kernel,
        mesh=plsc.VectorSubcoreMesh(core_axis_name="c", subcore_axis_name="s"),
        out_type=[
            jax.ShapeDtypeStruct((_NPAD, _CAP), jnp.float32),
            jax.ShapeDtypeStruct((_NPAD, _CAP), jnp.float32),
            jax.ShapeDtypeStruct((_NPAD, _CAP), jnp.float32),
            jax.ShapeDtypeStruct((_NPAD, _CAP), jnp.float32),
            jax.ShapeDtypeStruct((_NPAD, _CAP), jnp.int32),
        ],
        scratch_types=[
            pltpu.VMEM((_NPAD,), jnp.float32),
            pltpu.VMEM((_NPAD,), jnp.float32),
            pltpu.VMEM((_NPAD,), jnp.float32),
            pltpu.VMEM((_NPAD,), jnp.float32),
            pltpu.VMEM((8, _NPAD), jnp.float32),
            pltpu.VMEM((_CAP + 16,), jnp.float32),
            pltpu.VMEM((_CAP + 16,), jnp.int32),
            pltpu.VMEM((8, _CAP), jnp.float32),
            pltpu.VMEM((8, _CAP), jnp.float32),
            pltpu.VMEM((8, _CAP), jnp.float32),
            pltpu.VMEM((8, _CAP), jnp.float32),
            pltpu.VMEM((8, _CAP), jnp.int32),
        ],
        compiler_params=pltpu.CompilerParams(needs_layout_passes=False),
    )(_sc_compact)
    cd, cx, cy, cz, ci = sc_fn(d2m, tau1, kx, ky, kz)

    idx_out, val_out = pl.pallas_call(
        _sel_body,
        grid=(_NBLK,),
        in_specs=[
            pl.BlockSpec((_R, _CAP), lambda i: (i, 0)),
            pl.BlockSpec((_R, _CAP), lambda i: (i, 0)),
            pl.BlockSpec((_R, _CAP), lambda i: (i, 0)),
            pl.BlockSpec((_R, _CAP), lambda i: (i, 0)),
            pl.BlockSpec((_R, _CAP), lambda i: (i, 0)),
            pl.BlockSpec((_R, 128), lambda i: (i, 0)),
        ],
        out_specs=[
            pl.BlockSpec((_R, 128), lambda i: (i, 0)),
            pl.BlockSpec((_R, 128), lambda i: (i, 0)),
        ],
        out_shape=[
            jax.ShapeDtypeStruct((_NPAD, 128), jnp.int32),
            jax.ShapeDtypeStruct((_NPAD, 128), jnp.float32),
        ],
        scratch_shapes=[pltpu.VMEM((_R, _CAP), jnp.float32)],
        compiler_params=pltpu.CompilerParams(
            dimension_semantics=("arbitrary",)),
    )(cd, cx, cy, cz, ci, q_arr)

    attr_out = pl.pallas_call(
        _attr_body,
        grid=(_NBLK,),
        in_specs=[pl.BlockSpec((_R, 128), lambda i: (i, 0))],
        out_specs=pl.BlockSpec((8, _R, 128), lambda i: (0, i, 0)),
        out_shape=jax.ShapeDtypeStruct((8, _NPAD, 128), jnp.float32),
    )(val_out)

    nbr = idx_out[:_N_KEEP, :_K]
    src = nbr.reshape(-1)
    dst = jnp.repeat(jnp.arange(_N_KEEP), _K)
    row = jnp.concatenate([src, dst])
    col = jnp.concatenate([dst, src])
    edge_index = jnp.stack([row, col], axis=0)

    attr_half = jnp.transpose(
        attr_out[:_NUM_BINS, :_N_KEEP, :_K], (1, 2, 0)).reshape(-1, _NUM_BINS)
    edge_attr = jnp.concatenate([attr_half, attr_half], axis=0)
    return edge_index, edge_attr


# speedup vs baseline: 13.9671x; 2.0573x over previous
import functools

import jax
import jax.numpy as jnp
from jax import lax
from jax.experimental import pallas as pl
from jax.experimental.pallas import tpu as pltpu
from jax.experimental.pallas import tpu_sc as plsc

_N = 10000
_N_KEEP = 7000
_K = 50
_NUM_BINS = 5
_CUTOFF = 5.0
_RADIUS = 0.75

_NPAD = 7040
_R = 128
_NBLK = _NPAD // _R
_CAP = 128
_NSUB = 32
_ROWS_W = _NPAD // _NSUB
_NCHUNK = 110
_HCAP = 128


def _d2_body(keys_ref, q_ref, sq_ref, d2_ref, tau_ref, m64_ref):
    i = pl.program_id(0)
    sqq = q_ref[:, 3:4]
    sqk = sq_ref[0:1, :]
    mm = jnp.dot(q_ref[...], keys_ref[...],
                 preferred_element_type=jnp.float32)
    d2 = (sqq + sqk) - 2.0 * mm
    row = i * _R + lax.broadcasted_iota(jnp.int32, (_R, 1), 0)
    col = lax.broadcasted_iota(jnp.int32, (_R, _NPAD), 1)
    d2 = jnp.where(col == row, jnp.inf, d2)
    d2_ref[...] = d2
    lane = lax.broadcasted_iota(jnp.int32, (_R, 128), 1)
    cm = jnp.full((_R, 128), jnp.inf, jnp.float32)
    for c in range(_NCHUNK):
        mn = jnp.min(d2_ref[:, 64 * c:64 * (c + 1)], axis=1, keepdims=True)
        cm = jnp.where(lane == c, mn, cm)
    m64_ref[...] = cm
    lo = jnp.min(cm, axis=1, keepdims=True) - 1.0
    hi = jnp.max(jnp.where(cm < jnp.inf, cm, -jnp.inf), axis=1, keepdims=True)

    def bis(_, lohi):
        lo, hi = lohi
        t = 0.5 * (lo + hi)
        cnt = jnp.sum((cm <= t).astype(jnp.int32), axis=1, keepdims=True)
        ge = cnt >= _K
        return jnp.where(ge, lo, t), jnp.where(ge, t, hi)

    lo, hi = lax.fori_loop(0, 18, bis, (lo, hi))
    tau_ref[...] = jnp.broadcast_to(hi, (_R, 128))


def _sc_compact(d2_hbm, tau_hbm, m64_hbm, kx_hbm, ky_hbm, kz_hbm,
                od2_hbm, ox_hbm, oy_hbm, oz_hbm, oi_hbm,
                kxb, kyb, kzb, taub, rowb8, m64b, hcb, candd, candi,
                od2b, oxb, oyb, ozb, oib):
    wid = lax.axis_index("s") * 2 + lax.axis_index("c")
    g0 = 27 * wid + jnp.minimum(wid, 16)
    ng = jnp.where(wid < 16, 28, 27)
    pltpu.sync_copy(kx_hbm, kxb)
    pltpu.sync_copy(ky_hbm, kyb)
    pltpu.sync_copy(kz_hbm, kzb)
    pltpu.sync_copy(tau_hbm, taub)
    iota16 = lax.broadcasted_iota(jnp.int32, (16,), 0)

    def group_body(gl, _):
        g = g0 + gl
        pltpu.sync_copy(d2_hbm.at[pl.ds(g * 8, 8)], rowb8)
        pltpu.sync_copy(m64_hbm.at[pl.ds(g * 8, 8)], m64b)
        for rr in range(8):
            r = g * 8 + rr
            tauv = plsc.load_gather(taub, [jnp.full((16,), r, jnp.int32)])
            for s in range(_CAP // 16):
                candd[pl.ds(16 * s, 16)] = jnp.full((16,), jnp.inf,
                                                    jnp.float32)
                candi[pl.ds(16 * s, 16)] = jnp.zeros((16,), jnp.int32)

            offh = 0
            for gm in range(_NCHUNK // 16 + 1):
                mv = m64b[rr, pl.ds(16 * gm, 16)]
                hmask = mv <= tauv
                plsc.store_compressed(hcb.at[pl.ds(offh, 16)],
                                      iota16 + 16 * gm, mask=hmask)
                offh = jnp.minimum(offh + jnp.sum(hmask.astype(jnp.int32)),
                                   _HCAP)

            def hgroup(mz, off):
                hv = hcb[pl.ds(16 * mz, 16)]
                for t in range(16):
                    c = jnp.minimum(jnp.maximum(hv[t], 0), _NCHUNK - 1)
                    valid = (16 * mz + t) < offh
                    for s in range(4):
                        d2v = rowb8[rr, pl.ds(c * 64 + 16 * s, 16)]
                        mask = (d2v <= tauv) & valid
                        cnt = jnp.sum(mask.astype(jnp.int32))
                        iv = iota16 + (c * 64 + 16 * s)
                        plsc.store_compressed(candd.at[pl.ds(off, 16)],
                                              d2v, mask=mask)
                        plsc.store_compressed(candi.at[pl.ds(off, 16)],
                                              iv, mask=mask)
                        off = jnp.minimum(off + cnt, _CAP)
                return off

            nhg = (offh + 15) // 16
            lax.fori_loop(0, nhg, hgroup, 0)
            for s in range(_CAP // 16):
                sl = pl.ds(16 * s, 16)
                ivv = candi[sl]
                od2b[rr, sl] = candd[sl]
                oib[rr, sl] = ivv
                oxb[rr, sl] = plsc.load_gather(kxb, [ivv])
                oyb[rr, sl] = plsc.load_gather(kyb, [ivv])
                ozb[rr, sl] = plsc.load_gather(kzb, [ivv])
        sl8 = pl.ds(g * 8, 8)
        pltpu.sync_copy(od2b, od2_hbm.at[sl8])
        pltpu.sync_copy(oxb, ox_hbm.at[sl8])
        pltpu.sync_copy(oyb, oy_hbm.at[sl8])
        pltpu.sync_copy(ozb, oz_hbm.at[sl8])
        pltpu.sync_copy(oib, oi_hbm.at[sl8])
        return 0

    lax.fori_loop(0, ng, group_body, 0)


def _sel_body(cd_ref, cx_ref, cy_ref, cz_ref, ci_ref, keys_ref,
              idx_ref, val_ref, d2s):
    qx = keys_ref[0:1, :]
    qy = keys_ref[1:2, :]
    qz = keys_ref[2:3, :]
    dx = qx - cx_ref[...]
    dy = qy - cy_ref[...]
    dz = qz - cz_ref[...]
    e2 = dx * dx + dy * dy + dz * dz
    d2s[...] = cd_ref[...]
    idx_ref[...] = jnp.zeros((64, 128), jnp.int32)
    val_ref[...] = jnp.zeros((64, 128), jnp.float32)
    cidx = lax.broadcasted_iota(jnp.int32, (_CAP, 128), 0)
    kidx = lax.broadcasted_iota(jnp.int32, (64, 128), 0)

    def body(k, _):
        d2c = d2s[...]
        m = jnp.min(d2c, axis=0, keepdims=True)
        sel = jnp.min(jnp.where(d2c == m, cidx, _CAP), axis=0, keepdims=True)
        hit = cidx == sel
        ev = jnp.min(jnp.where(hit, e2, jnp.inf), axis=0, keepdims=True)
        gi = jnp.min(jnp.where(hit, ci_ref[...], _NPAD), axis=0, keepdims=True)
        val_ref[...] = jnp.where(kidx == k, ev, val_ref[...])
        idx_ref[...] = jnp.where(kidx == k, gi, idx_ref[...])
        d2s[...] = jnp.where(hit, jnp.inf, d2c)
        return 0

    lax.fori_loop(0, _K, body, 0)


def _attr_body(d2_ref, attr_ref):
    centers = [_CUTOFF / (_NUM_BINS - 1) * c for c in range(_NUM_BINS)]
    sigma = centers[1] - centers[0]
    inv = 1.0 / (2.0 * sigma * sigma)
    dist = jnp.sqrt(d2_ref[...] + 1e-12)
    for c in range(_NUM_BINS):
        diff = dist - centers[c]
        attr_ref[c] = jnp.exp(-(diff * diff) * inv)
    for c in range(_NUM_BINS, 8):
        attr_ref[c] = jnp.zeros_like(dist)


def kernel(pos):
    base = jax.random.key(1)
    k1 = jax.random.fold_in(base, 0)
    k2 = jax.random.fold_in(base, 1)
    k3 = jax.random.fold_in(base, 2)
    scores = jax.random.uniform(k1, (_N,))
    keep_idx = jnp.argsort(scores)[:_N_KEEP]
    p = jnp.take(pos, keep_idx, axis=0)
    dirs = jax.random.normal(k2, (_N_KEEP, 3), dtype=jnp.float32)
    dirs = dirs / (jnp.linalg.norm(dirs, axis=1, keepdims=True) + 1e-12)
    u = jax.random.uniform(k3, (_N_KEEP, 1), dtype=jnp.float32)
    p = p + dirs * _RADIUS * (u ** (1.0 / 3.0))

    sq = jnp.sum(p * p, axis=1)

    p_pad = jnp.pad(p, ((0, _NPAD - _N_KEEP), (0, 0)))
    q_arr = jnp.concatenate(
        [p_pad, jnp.pad(sq, (0, _NPAD - _N_KEEP))[:, None],
         jnp.zeros((_NPAD, 128 - 4), jnp.float32)], axis=1)
    keys = jnp.pad(p_pad.T, ((0, 128 - 3), (0, 0)))
    sq_in = jnp.pad(
        jnp.pad(sq, (0, _NPAD - _N_KEEP),
                constant_values=jnp.inf)[None, :], ((0, 7), (0, 0)))

    d2m, tau, m64 = pl.pallas_call(
        _d2_body,
        grid=(_NBLK,),
        in_specs=[
            pl.BlockSpec((128, _NPAD), lambda i: (0, 0)),
            pl.BlockSpec((_R, 128), lambda i: (i, 0)),
            pl.BlockSpec((8, _NPAD), lambda i: (0, 0)),
        ],
        out_specs=[
            pl.BlockSpec((_R, _NPAD), lambda i: (i, 0)),
            pl.BlockSpec((_R, 128), lambda i: (i, 0)),
            pl.BlockSpec((_R, 128), lambda i: (i, 0)),
        ],
        out_shape=[
            jax.ShapeDtypeStruct((_NPAD, _NPAD), jnp.float32),
            jax.ShapeDtypeStruct((_NPAD, 128), jnp.float32),
            jax.ShapeDtypeStruct((_NPAD, 128), jnp.float32),
        ],
        compiler_params=pltpu.CompilerParams(
            dimension_semantics=("arbitrary",)),
    )(keys, q_arr, sq_in)

    kx = p_pad[:, 0]
    ky = p_pad[:, 1]
    kz = p_pad[:, 2]
    tau1 = tau[:, 0]

    sc_fn = functools.partial(
        pl.kernel,
        mesh=plsc.VectorSubcoreMesh(core_axis_name="c", subcore_axis_name="s"),
        out_type=[
            jax.ShapeDtypeStruct((_NPAD, _CAP), jnp.float32),
            jax.ShapeDtypeStruct((_NPAD, _CAP), jnp.float32),
            jax.ShapeDtypeStruct((_NPAD, _CAP), jnp.float32),
            jax.ShapeDtypeStruct((_NPAD, _CAP), jnp.float32),
            jax.ShapeDtypeStruct((_NPAD, _CAP), jnp.int32),
        ],
        scratch_types=[
            pltpu.VMEM((_NPAD,), jnp.float32),
            pltpu.VMEM((_NPAD,), jnp.float32),
            pltpu.VMEM((_NPAD,), jnp.float32),
            pltpu.VMEM((_NPAD,), jnp.float32),
            pltpu.VMEM((8, _NPAD), jnp.float32),
            pltpu.VMEM((8, 128), jnp.float32),
            pltpu.VMEM((_HCAP + 16,), jnp.int32),
            pltpu.VMEM((_CAP + 16,), jnp.float32),
            pltpu.VMEM((_CAP + 16,), jnp.int32),
            pltpu.VMEM((8, _CAP), jnp.float32),
            pltpu.VMEM((8, _CAP), jnp.float32),
            pltpu.VMEM((8, _CAP), jnp.float32),
            pltpu.VMEM((8, _CAP), jnp.float32),
            pltpu.VMEM((8, _CAP), jnp.int32),
        ],
        compiler_params=pltpu.CompilerParams(needs_layout_passes=False),
    )(_sc_compact)
    cd, cx, cy, cz, ci = sc_fn(d2m, tau1, m64, kx, ky, kz)
    cdT = cd.T
    cxT = cx.T
    cyT = cy.T
    czT = cz.T
    ciT = ci.T

    idx_out, val_out = pl.pallas_call(
        _sel_body,
        grid=(_NBLK,),
        in_specs=[
            pl.BlockSpec((_CAP, 128), lambda i: (0, i)),
            pl.BlockSpec((_CAP, 128), lambda i: (0, i)),
            pl.BlockSpec((_CAP, 128), lambda i: (0, i)),
            pl.BlockSpec((_CAP, 128), lambda i: (0, i)),
            pl.BlockSpec((_CAP, 128), lambda i: (0, i)),
            pl.BlockSpec((8, 128), lambda i: (0, i)),
        ],
        out_specs=[
            pl.BlockSpec((64, 128), lambda i: (0, i)),
            pl.BlockSpec((64, 128), lambda i: (0, i)),
        ],
        out_shape=[
            jax.ShapeDtypeStruct((64, _NPAD), jnp.int32),
            jax.ShapeDtypeStruct((64, _NPAD), jnp.float32),
        ],
        scratch_shapes=[pltpu.VMEM((_CAP, 128), jnp.float32)],
        compiler_params=pltpu.CompilerParams(
            dimension_semantics=("arbitrary",)),
    )(cdT, cxT, cyT, czT, ciT, keys[:8])

    attr_out = pl.pallas_call(
        _attr_body,
        grid=(_NBLK,),
        in_specs=[pl.BlockSpec((64, 128), lambda i: (0, i))],
        out_specs=pl.BlockSpec((8, 64, 128), lambda i: (0, 0, i)),
        out_shape=jax.ShapeDtypeStruct((8, 64, _NPAD), jnp.float32),
    )(val_out)

    nbr = idx_out[:_K, :_N_KEEP].T
    src = nbr.reshape(-1)
    dst = jnp.repeat(jnp.arange(_N_KEEP), _K)
    row = jnp.concatenate([src, dst])
    col = jnp.concatenate([dst, src])
    edge_index = jnp.stack([row, col], axis=0)

    attr_half = jnp.transpose(
        attr_out[:_NUM_BINS, :_K, :_N_KEEP], (2, 1, 0)).reshape(-1, _NUM_BINS)
    edge_attr = jnp.concatenate([attr_half, attr_half], axis=0)
    return edge_index, edge_attr
